# baseline (speedup 1.0000x reference)
import jax
import jax.numpy as jnp
from jax import lax
from jax.experimental import pallas as pl
from jax.experimental.pallas import tpu as pltpu
from jax.experimental.pallas import tpu_sc as plsc

D = 64
GRP = 128
NC, NS = 2, 16
NW = NC * NS
GPB = 4
CH = GPB * GRP
NBUF = 3

PBLK = 2000


def _pack_body(t_ref, o_ref):
    o_ref[...] = jnp.concatenate([t_ref[0:PBLK], t_ref[PBLK:]], axis=1)


def _gather_body(x_hbm, table_hbm, out_hbm, idx_v,
                 rows0, rows1, rows2, sg0, sg1, sg2, so0, so1, so2):
    wid = lax.axis_index("s") * NC + lax.axis_index("c")
    gw = x_hbm.shape[0] // NW
    n_chunks = gw // GPB
    row_base = wid * gw * GRP

    pltpu.sync_copy(x_hbm.at[pl.ds(wid * gw, gw)], idx_v)

    rows = (rows0, rows1, rows2)
    sg = (sg0, sg1, sg2)
    so = (so0, so1, so2)

    def issue_gathers(g, b):
        for j in range(GPB):
            pltpu.async_copy(
                table_hbm.at[idx_v.at[g * GPB + j]],
                rows[b].at[pl.ds(j * GRP, GRP)],
                sg[b])

    def drain_gathers(b):
        for j in range(GPB):
            pltpu.make_async_copy(
                table_hbm.at[idx_v.at[0]],
                rows[b].at[pl.ds(j * GRP, GRP)],
                sg[b]).wait()

    def issue_writeout(g, b):
        pltpu.async_copy(rows[b], out_hbm.at[pl.ds(row_base + g * CH, CH)],
                         so[b])

    def drain_writeout(b):
        pltpu.make_async_copy(rows[b], out_hbm.at[pl.ds(row_base, CH)],
                              so[b]).wait()

    def step(g, b, wait_prev=True, issue_next=True):
        drain_gathers(b)
        issue_writeout(g, b)
        if issue_next:
            bn = (b + 2) % NBUF
            if wait_prev:
                drain_writeout(bn)
            issue_gathers(g + 2, bn)

    issue_gathers(0, 0)
    issue_gathers(1, 1)
    step(0, 0, wait_prev=False)

    steady = (n_chunks - 3) // NBUF

    @pl.loop(0, steady)
    def _steady(t):
        for k in range(NBUF):
            g = 1 + t * NBUF + k
            step(g, (1 + k) % NBUF)

    for g in range(1 + steady * NBUF, n_chunks - 2):
        step(g, g % NBUF)
    for g in range(n_chunks - 2, n_chunks):
        step(g, g % NBUF, issue_next=False)

    for b in range(NBUF):
        drain_writeout(b)


def kernel(x, table):
    B, H = x.shape
    BT = B * H
    V = table.shape[0]

    xi = x.astype(jnp.int32)
    m = xi % (2 * PBLK)
    xp = xi - m + jnp.where(m < PBLK, 2 * m, 2 * m - (2 * PBLK - 1))
    xf = xp.reshape(BT // GRP, GRP)

    pack = pl.pallas_call(
        _pack_body,
        out_shape=jax.ShapeDtypeStruct((V // 2, 2 * D), jnp.float32),
        grid=(V // 2 // PBLK,),
        in_specs=[pl.BlockSpec((2 * PBLK, D), lambda i: (i, 0))],
        out_specs=pl.BlockSpec((PBLK, 2 * D), lambda i: (i, 0)),
    )
    table_lin = pack(table).reshape(V, D)

    gather = pl.kernel(
        _gather_body,
        out_type=jax.ShapeDtypeStruct((BT, D), jnp.float32),
        mesh=plsc.VectorSubcoreMesh(core_axis_name="c", subcore_axis_name="s"),
        compiler_params=pltpu.CompilerParams(use_tc_tiling_on_sc=False),
        scratch_types=[
            pltpu.VMEM((BT // GRP // NW, GRP), jnp.int32),
            pltpu.VMEM((CH, D), jnp.float32),
            pltpu.VMEM((CH, D), jnp.float32),
            pltpu.VMEM((CH, D), jnp.float32),
            pltpu.SemaphoreType.DMA,
            pltpu.SemaphoreType.DMA,
            pltpu.SemaphoreType.DMA,
            pltpu.SemaphoreType.DMA,
            pltpu.SemaphoreType.DMA,
            pltpu.SemaphoreType.DMA,
        ],
    )
    out = gather(xf, table_lin)
    return out.reshape(B, H, D)


# speedup vs baseline: 1.0502x; 1.0502x over previous
import jax
import jax.numpy as jnp
from jax import lax
from jax.experimental import pallas as pl
from jax.experimental.pallas import tpu as pltpu
from jax.experimental.pallas import tpu_sc as plsc

D = 64
GRP = 128
NC, NS = 2, 16
NW = NC * NS
GPB = 4
CH = GPB * GRP
NBUF = 3


def _gather_body(x_hbm, table_hbm, out_hbm, idx_v,
                 rows0, rows1, rows2, sg0, sg1, sg2, so0, so1, so2):
    wid = lax.axis_index("s") * NC + lax.axis_index("c")
    gw = x_hbm.shape[0] // NW
    n_chunks = gw // GPB
    row_base = wid * gw * GRP

    pltpu.sync_copy(x_hbm.at[pl.ds(wid * gw, gw)], idx_v)

    rows = (rows0, rows1, rows2)
    sg = (sg0, sg1, sg2)
    so = (so0, so1, so2)

    def issue_gathers(g, b):
        for j in range(GPB):
            pltpu.async_copy(
                table_hbm.at[idx_v.at[g * GPB + j]],
                rows[b].at[pl.ds(j * GRP, GRP)],
                sg[b])

    def drain_gathers(b):
        for j in range(GPB):
            pltpu.make_async_copy(
                table_hbm.at[idx_v.at[0]],
                rows[b].at[pl.ds(j * GRP, GRP)],
                sg[b]).wait()

    def issue_writeout(g, b):
        pltpu.async_copy(rows[b], out_hbm.at[pl.ds(row_base + g * CH, CH)],
                         so[b])

    def drain_writeout(b):
        pltpu.make_async_copy(rows[b], out_hbm.at[pl.ds(row_base, CH)],
                              so[b]).wait()

    def step(g, b, wait_prev=True, issue_next=True):
        drain_gathers(b)
        issue_writeout(g, b)
        if issue_next:
            bn = (b + 2) % NBUF
            if wait_prev:
                drain_writeout(bn)
            issue_gathers(g + 2, bn)

    issue_gathers(0, 0)
    issue_gathers(1, 1)
    step(0, 0, wait_prev=False)

    steady = (n_chunks - 3) // NBUF

    @pl.loop(0, steady)
    def _steady(t):
        for k in range(NBUF):
            g = 1 + t * NBUF + k
            step(g, (1 + k) % NBUF)

    for g in range(1 + steady * NBUF, n_chunks - 2):
        step(g, g % NBUF)
    for g in range(n_chunks - 2, n_chunks):
        step(g, g % NBUF, issue_next=False)

    for b in range(NBUF):
        drain_writeout(b)


def kernel(x, table):
    B, H = x.shape
    BT = B * H
    xf = x.astype(jnp.int32).reshape(BT // GRP, GRP)
    gather = pl.kernel(
        _gather_body,
        out_type=jax.ShapeDtypeStruct((BT, D), jnp.float32),
        mesh=plsc.VectorSubcoreMesh(core_axis_name="c", subcore_axis_name="s"),
        compiler_params=pltpu.CompilerParams(use_tc_tiling_on_sc=False),
        scratch_types=[
            pltpu.VMEM((BT // GRP // NW, GRP), jnp.int32),
            pltpu.VMEM((CH, D), jnp.float32),
            pltpu.VMEM((CH, D), jnp.float32),
            pltpu.VMEM((CH, D), jnp.float32),
            pltpu.SemaphoreType.DMA,
            pltpu.SemaphoreType.DMA,
            pltpu.SemaphoreType.DMA,
            pltpu.SemaphoreType.DMA,
            pltpu.SemaphoreType.DMA,
            pltpu.SemaphoreType.DMA,
        ],
    )
    out = gather(xf, table)
    return out.reshape(B, H, D)
